# baseline (speedup 1.0000x reference)
import functools

import jax
import jax.numpy as jnp
from jax import lax
from jax.experimental import pallas as pl
from jax.experimental.pallas import tpu as pltpu
from jax.experimental.pallas import tpu_sc as plsc

VOCAB = 1000000
EDIM = 32
BATCH = 4096
HLEN = 50
HID = 512
OUT = 1000

NC, NS = 2, 16
NW = NC * NS
SAMPLES_PER_W = BATCH // NW
CHUNK_SAMPLES = 2
CHUNK_IDX = CHUNK_SAMPLES * HLEN
CHUNKS_PER_W = SAMPLES_PER_W // CHUNK_SAMPLES
ROWS_PER_BLOCK = 4


def _embed_body(q_hbm, off_hbm, tbl_hbm, out_hbm,
                q_v, off_v, buf0, buf1, out_v, sem0, sem1):
    wid = lax.axis_index("s") * NC + lax.axis_index("c")
    idx_base = wid * CHUNKS_PER_W
    pltpu.sync_copy(q_hbm.at[pl.ds(idx_base, CHUNKS_PER_W)], q_v)
    pltpu.sync_copy(off_hbm.at[pl.ds(idx_base, CHUNKS_PER_W)], off_v)

    inv = jnp.float32(1.0 / HLEN)
    bufs = (buf0, buf1)
    sems = (sem0, sem1)

    cps = [None, None]
    cps[0] = pltpu.async_copy(tbl_hbm.at[q_v.at[0]], bufs[0], sems[0])
    for c in range(CHUNKS_PER_W):
        b = c % 2
        if c + 1 < CHUNKS_PER_W:
            nb = (c + 1) % 2
            cps[nb] = pltpu.async_copy(tbl_hbm.at[q_v.at[c + 1]], bufs[nb], sems[nb])
        cps[b].wait()
        buf = bufs[b]

        def body(j, accs):
            a0, a1, a2, a3 = accs
            oa = off_v[c, pl.ds(j, 16)][0]
            ob = off_v[c, pl.ds(HLEN + j, 16)][0]
            return (a0 + buf[j, pl.ds(oa, 16)],
                    a1 + buf[j, pl.ds(oa + 16, 16)],
                    a2 + buf[HLEN + j, pl.ds(ob, 16)],
                    a3 + buf[HLEN + j, pl.ds(ob + 16, 16)])

        z = jnp.zeros((16,), jnp.float32)
        a0, a1, a2, a3 = lax.fori_loop(0, HLEN, body, (z, z, z, z))
        out_v[2 * c, 0:16] = a0 * inv
        out_v[2 * c, 16:32] = a1 * inv
        out_v[2 * c + 1, 0:16] = a2 * inv
        out_v[2 * c + 1, 16:32] = a3 * inv

    pltpu.sync_copy(out_v, out_hbm.at[pl.ds(wid * SAMPLES_PER_W, SAMPLES_PER_W)])


_embed = functools.partial(
    pl.kernel,
    out_type=jax.ShapeDtypeStruct((BATCH, EDIM), jnp.float32),
    mesh=plsc.VectorSubcoreMesh(core_axis_name="c", subcore_axis_name="s"),
    scratch_types=[
        pltpu.VMEM((CHUNKS_PER_W, CHUNK_IDX), jnp.int32),
        pltpu.VMEM((CHUNKS_PER_W, 128), jnp.int32),
        pltpu.VMEM((CHUNK_IDX, ROWS_PER_BLOCK * EDIM), jnp.float32),
        pltpu.VMEM((CHUNK_IDX, ROWS_PER_BLOCK * EDIM), jnp.float32),
        pltpu.VMEM((SAMPLES_PER_W, EDIM), jnp.float32),
        pltpu.SemaphoreType.DMA,
        pltpu.SemaphoreType.DMA,
    ],
    compiler_params=pltpu.CompilerParams(use_tc_tiling_on_sc=False),
)(_embed_body)


def _embed_table(idx, E):
    q = lax.shift_right_logical(idx, 2).reshape(-1, CHUNK_IDX)
    off = lax.shift_left(lax.bitwise_and(idx, 3), 5).reshape(-1, CHUNK_IDX)
    off = jnp.pad(off, ((0, 0), (0, 128 - CHUNK_IDX)))
    tbl = jnp.concatenate([E[m::ROWS_PER_BLOCK] for m in range(ROWS_PER_BLOCK)],
                          axis=1)
    return _embed(q, off, tbl)


def _mlp_body(x0_ref, x1_ref, w0_ref, b0_ref, w1_ref, b1_ref,
              wc_ref, bc_ref, wk_ref, bk_ref, outc_ref, outk_ref):
    dn = (((1,), (1,)), ((), ()))
    x = jnp.concatenate([x0_ref[...], x1_ref[...]], axis=1)
    h = lax.dot_general(x, w0_ref[...], dn,
                        preferred_element_type=jnp.float32) + b0_ref[...]
    h = jnp.maximum(h, 0.0)
    h = lax.dot_general(h, w1_ref[...], dn,
                        preferred_element_type=jnp.float32) + b1_ref[...]
    h = jnp.maximum(h, 0.0)
    outc_ref[...] = lax.dot_general(h, wc_ref[...], dn,
                                    preferred_element_type=jnp.float32) + bc_ref[...]
    outk_ref[...] = lax.dot_general(h, wk_ref[...], dn,
                                    preferred_element_type=jnp.float32) + bk_ref[...]


_MLP_BLOCK = 512


def _mlp(x0, x1, W0, b0, W1, b1, Wc, bc, Wk, bk):
    grid = (BATCH // _MLP_BLOCK,)
    fixed = lambda i: (0, 0)
    return pl.pallas_call(
        _mlp_body,
        grid=grid,
        in_specs=[
            pl.BlockSpec((_MLP_BLOCK, EDIM), lambda i: (i, 0)),
            pl.BlockSpec((_MLP_BLOCK, EDIM), lambda i: (i, 0)),
            pl.BlockSpec((HID, 2 * EDIM), fixed),
            pl.BlockSpec((1, HID), fixed),
            pl.BlockSpec((HID, HID), fixed),
            pl.BlockSpec((1, HID), fixed),
            pl.BlockSpec((OUT, HID), fixed),
            pl.BlockSpec((1, OUT), fixed),
            pl.BlockSpec((OUT, HID), fixed),
            pl.BlockSpec((1, OUT), fixed),
        ],
        out_specs=[
            pl.BlockSpec((_MLP_BLOCK, OUT), lambda i: (i, 0)),
            pl.BlockSpec((_MLP_BLOCK, OUT), lambda i: (i, 0)),
        ],
        out_shape=[
            jax.ShapeDtypeStruct((BATCH, OUT), jnp.float32),
            jax.ShapeDtypeStruct((BATCH, OUT), jnp.float32),
        ],
    )(x0, x1, W0, b0, W1, b1, Wc, bc, Wk, bk)


def kernel(idx0, idx1, E0, E1, W0, b0, W1, b1, Wc, bc, Wk, bk):
    x0 = _embed_table(idx0.astype(jnp.int32), E0)
    x1 = _embed_table(idx1.astype(jnp.int32), E1)
    outc, outk = _mlp(x0, x1, W0, b0.reshape(1, HID), W1, b1.reshape(1, HID),
                      Wc, bc.reshape(1, OUT), Wk, bk.reshape(1, OUT))
    return (outc, outk)


# speedup vs baseline: 9.0988x; 9.0988x over previous
import functools

import jax
import jax.numpy as jnp
from jax import lax
from jax.experimental import pallas as pl
from jax.experimental.pallas import tpu as pltpu
from jax.experimental.pallas import tpu_sc as plsc

VOCAB = 1000000
EDIM = 32
BATCH = 4096
HLEN = 50
HID = 512
OUT = 1000

NC, NS = 2, 16
NW = NC * NS
SAMPLES_PER_W = BATCH // NW
CHUNK_SAMPLES = 2
CHUNK_IDX = CHUNK_SAMPLES * HLEN
CHUNKS_PER_W = SAMPLES_PER_W // CHUNK_SAMPLES
IDX_ROWS_PER_W = (SAMPLES_PER_W * HLEN) // CHUNK_IDX


def _embed_body(idx0_hbm, idx1_hbm, e0_hbm, e1_hbm, out_hbm,
                idx_v0, idx_v1, buf0, buf1, out_v, sem0, sem1):
    wid = lax.axis_index("s") * NC + lax.axis_index("c")
    idx_base = wid * IDX_ROWS_PER_W
    pltpu.sync_copy(idx0_hbm.at[pl.ds(idx_base, IDX_ROWS_PER_W)], idx_v0)
    pltpu.sync_copy(idx1_hbm.at[pl.ds(idx_base, IDX_ROWS_PER_W)], idx_v1)

    inv = jnp.float32(1.0 / HLEN)
    bufs = (buf0, buf1)
    sems = (sem0, sem1)

    def table_loop(table_hbm, idxv, col):
        cps = [None, None]
        cps[0] = pltpu.async_copy(table_hbm.at[idxv.at[0]], bufs[0], sems[0])
        for c in range(CHUNKS_PER_W):
            b = c % 2
            if c + 1 < CHUNKS_PER_W:
                nb = (c + 1) % 2
                cps[nb] = pltpu.async_copy(
                    table_hbm.at[idxv.at[c + 1]], bufs[nb], sems[nb])
            cps[b].wait()
            buf = bufs[b]

            def body(j, accs):
                a0, a1, a2, a3 = accs
                return (a0 + buf[j, 0:16],
                        a1 + buf[j, 16:32],
                        a2 + buf[HLEN + j, 0:16],
                        a3 + buf[HLEN + j, 16:32])

            z = jnp.zeros((16,), jnp.float32)
            a0, a1, a2, a3 = lax.fori_loop(0, HLEN, body, (z, z, z, z))
            out_v[2 * c, col:col + 16] = a0 * inv
            out_v[2 * c, col + 16:col + 32] = a1 * inv
            out_v[2 * c + 1, col:col + 16] = a2 * inv
            out_v[2 * c + 1, col + 16:col + 32] = a3 * inv

    table_loop(e0_hbm, idx_v0, 0)
    table_loop(e1_hbm, idx_v1, EDIM)

    pltpu.sync_copy(out_v, out_hbm.at[pl.ds(wid * SAMPLES_PER_W, SAMPLES_PER_W)])


_embed = functools.partial(
    pl.kernel,
    out_type=jax.ShapeDtypeStruct((BATCH, 2 * EDIM), jnp.float32),
    mesh=plsc.VectorSubcoreMesh(core_axis_name="c", subcore_axis_name="s"),
    scratch_types=[
        pltpu.VMEM((IDX_ROWS_PER_W, CHUNK_IDX), jnp.int32),
        pltpu.VMEM((IDX_ROWS_PER_W, CHUNK_IDX), jnp.int32),
        pltpu.VMEM((CHUNK_IDX, EDIM), jnp.float32),
        pltpu.VMEM((CHUNK_IDX, EDIM), jnp.float32),
        pltpu.VMEM((SAMPLES_PER_W, 2 * EDIM), jnp.float32),
        pltpu.SemaphoreType.DMA,
        pltpu.SemaphoreType.DMA,
    ],
    compiler_params=pltpu.CompilerParams(use_tc_tiling_on_sc=False),
)(_embed_body)


def _mlp_body(x_ref, w0_ref, b0_ref, w1_ref, b1_ref,
              wc_ref, bc_ref, wk_ref, bk_ref, outc_ref, outk_ref):
    dn = (((1,), (1,)), ((), ()))
    x = x_ref[...]
    h = lax.dot_general(x, w0_ref[...], dn,
                        preferred_element_type=jnp.float32) + b0_ref[...]
    h = jnp.maximum(h, 0.0)
    h = lax.dot_general(h, w1_ref[...], dn,
                        preferred_element_type=jnp.float32) + b1_ref[...]
    h = jnp.maximum(h, 0.0)
    outc_ref[...] = lax.dot_general(h, wc_ref[...], dn,
                                    preferred_element_type=jnp.float32) + bc_ref[...]
    outk_ref[...] = lax.dot_general(h, wk_ref[...], dn,
                                    preferred_element_type=jnp.float32) + bk_ref[...]


_MLP_BLOCK = 512


def _mlp(x, W0, b0, W1, b1, Wc, bc, Wk, bk):
    grid = (BATCH // _MLP_BLOCK,)
    fixed = lambda i: (0, 0)
    return pl.pallas_call(
        _mlp_body,
        grid=grid,
        in_specs=[
            pl.BlockSpec((_MLP_BLOCK, 2 * EDIM), lambda i: (i, 0)),
            pl.BlockSpec((HID, 2 * EDIM), fixed),
            pl.BlockSpec((1, HID), fixed),
            pl.BlockSpec((HID, HID), fixed),
            pl.BlockSpec((1, HID), fixed),
            pl.BlockSpec((OUT, HID), fixed),
            pl.BlockSpec((1, OUT), fixed),
            pl.BlockSpec((OUT, HID), fixed),
            pl.BlockSpec((1, OUT), fixed),
        ],
        out_specs=[
            pl.BlockSpec((_MLP_BLOCK, OUT), lambda i: (i, 0)),
            pl.BlockSpec((_MLP_BLOCK, OUT), lambda i: (i, 0)),
        ],
        out_shape=[
            jax.ShapeDtypeStruct((BATCH, OUT), jnp.float32),
            jax.ShapeDtypeStruct((BATCH, OUT), jnp.float32),
        ],
    )(x, W0, b0, W1, b1, Wc, bc, Wk, bk)


def kernel(idx0, idx1, E0, E1, W0, b0, W1, b1, Wc, bc, Wk, bk):
    idx0_r = idx0.astype(jnp.int32).reshape(-1, CHUNK_IDX)
    idx1_r = idx1.astype(jnp.int32).reshape(-1, CHUNK_IDX)
    x = _embed(idx0_r, idx1_r, E0, E1)
    outc, outk = _mlp(x, W0, b0.reshape(1, HID), W1, b1.reshape(1, HID),
                      Wc, bc.reshape(1, OUT), Wk, bk.reshape(1, OUT))
    return (outc, outk)
